# baseline (speedup 1.0000x reference)
import functools

import jax
import jax.numpy as jnp
from jax import lax
from jax.experimental import pallas as pl
from jax.experimental.pallas import tpu as pltpu
from jax.experimental.pallas import tpu_sc as plsc

B = 16384
D = 32
NUM_TABLES = 26
NUM = 13
NUM_PAD = 16
H = 256
XW = 896
NG = XW // 128
CHUNK = 64
NCHUNK = 8
SLAB = 4 * CHUNK * D

_info = plsc.get_sparse_core_info()
NC = _info.num_cores
NS = _info.num_subcores
NW = NC * NS
BPW = B // NW

_COL = [32 + 32 * i for i in range(NUM_TABLES)]
_GROUPS = [[i for i in range(NUM_TABLES) if _COL[i] // 128 == g]
           for g in range(NG)]


def _sc_body(zeros_hbm, num_hbm, idxe_hbm, *rest):
    tables = rest[:NUM_TABLES]
    x_out = rest[NUM_TABLES]
    staging = rest[NUM_TABLES + 1]
    idxs = rest[NUM_TABLES + 2: NUM_TABLES + 6]
    bfs = rest[NUM_TABLES + 6: NUM_TABLES + 8]
    nfat = rest[NUM_TABLES + 8]
    sems, sem_i, sem_n, sem_w = rest[NUM_TABLES + 9:]

    wid = lax.axis_index("s") * NC + lax.axis_index("c")
    base = wid * BPW

    pltpu.sync_copy(zeros_hbm, staging)

    def fire(i, ti, slot):
        pltpu.async_copy(
            tables[i].at[idxs[ti]],
            bfs[slot], sems.at[slot])

    def drain_extract(i, ti, slot, p):
        col = _COL[i] % 128
        pltpu.make_async_copy(
            tables[i].at[idxs[0]],
            bfs[slot], sems.at[slot]).wait()

        def blk(k, _):
            for j in range(16):
                r = 16 * k + j
                for h in range(D // 16):
                    staging[p, r, pl.ds(col + 16 * h, 16)] = (
                        bfs[slot][pl.ds(D * r + 16 * h, 16)])
            return _

        lax.fori_loop(0, CHUNK // 16, blk, None)

    for g in range(NG):
        grp = _GROUPS[g]

        def chunk_body(c, _, g=g, grp=grp):
            p = lax.rem(c, 2)
            for ti in range(len(grp)):
                pltpu.make_async_copy(
                    idxe_hbm.at[wid, g, 0, ti], idxs[ti], sem_i).wait()

            for ti in range(min(2, len(grp))):
                fire(grp[ti], ti, ti % 2)
            if g == 0:
                noff = pl.multiple_of((base + CHUNK * c) // 8, 8)
                pltpu.async_copy(num_hbm.at[pl.ds(noff, CHUNK // 8)],
                                 nfat, sem_n)

            @pl.when(c >= 2)
            def _wait_prev():
                pltpu.make_async_copy(
                    staging.at[p],
                    x_out.at[pl.ds(base, CHUNK), pl.ds(128 * g, 128)],
                    sem_w.at[p]).wait()

            for ti, i in enumerate(grp):
                drain_extract(i, ti, ti % 2, p)
                if ti + 2 < len(grp):
                    fire(grp[ti + 2], ti + 2, (ti + 2) % 2)

            @pl.when(c < NCHUNK - 1)
            def _prefetch():
                for ti in range(len(grp)):
                    pltpu.async_copy(idxe_hbm.at[wid, g, c + 1, ti],
                                     idxs[ti], sem_i)

            if g == 0:
                pltpu.make_async_copy(
                    num_hbm.at[pl.ds(0, CHUNK // 8)], nfat, sem_n).wait()

                def nblk(k, _):
                    for j in range(16):
                        r = 16 * k + j
                        staging[p, r, pl.ds(0, NUM_PAD)] = (
                            nfat[r // 8, pl.ds(NUM_PAD * (r % 8), NUM_PAD)])
                    return _

                lax.fori_loop(0, CHUNK // 16, nblk, None)

            xoff = pl.multiple_of(base + CHUNK * c, CHUNK)
            pltpu.async_copy(
                staging.at[p],
                x_out.at[pl.ds(xoff, CHUNK), pl.ds(128 * g, 128)],
                sem_w.at[p])
            return _

        for ti in range(len(grp)):
            pltpu.async_copy(idxe_hbm.at[wid, g, 0, ti], idxs[ti], sem_i)
        lax.fori_loop(0, NCHUNK, chunk_body, None)
        for p in range(2):
            pltpu.make_async_copy(
                staging.at[p],
                x_out.at[pl.ds(base, CHUNK), pl.ds(128 * g, 128)],
                sem_w.at[p]).wait()


_sc_fill = functools.partial(
    pl.kernel,
    mesh=plsc.VectorSubcoreMesh(core_axis_name="c", subcore_axis_name="s"),
    out_type=jax.ShapeDtypeStruct((B, XW), jnp.float32),
    scratch_types=(
        [pltpu.VMEM((2, CHUNK, 128), jnp.float32)]
        + [pltpu.VMEM((2048,), jnp.int32) for _ in range(4)]
        + [pltpu.VMEM((CHUNK * D,), jnp.float32) for _ in range(2)]
        + [pltpu.VMEM((CHUNK // 8, 128), jnp.float32),
           pltpu.SemaphoreType.DMA((2,)),
           pltpu.SemaphoreType.DMA,
           pltpu.SemaphoreType.DMA,
           pltpu.SemaphoreType.DMA((2,))]
    ),
)(_sc_body)


TB = 1024


def _mm_body(x_ref, w_ref, b_ref, o_ref):
    o_ref[...] = (
        jnp.dot(x_ref[...], w_ref[...], preferred_element_type=jnp.float32)
        + b_ref[...]
    )


_tc_matmul = pl.pallas_call(
    _mm_body,
    grid=(B // TB,),
    in_specs=[
        pl.BlockSpec((TB, XW), lambda i: (i, 0)),
        pl.BlockSpec((XW, H), lambda i: (0, 0)),
        pl.BlockSpec((1, H), lambda i: (0, 0)),
    ],
    out_specs=pl.BlockSpec((TB, H), lambda i: (i, 0)),
    out_shape=jax.ShapeDtypeStruct((B, H), jnp.float32),
)


def kernel(numerical, cat_0, cat_1, cat_2, cat_3, cat_4, cat_5, cat_6, cat_7, cat_8, cat_9, cat_10, cat_11, cat_12, cat_13, cat_14, cat_15, cat_16, cat_17, cat_18, cat_19, cat_20, cat_21, cat_22, cat_23, cat_24, cat_25, emb_0, emb_1, emb_2, emb_3, emb_4, emb_5, emb_6, emb_7, emb_8, emb_9, emb_10, emb_11, emb_12, emb_13, emb_14, emb_15, emb_16, emb_17, emb_18, emb_19, emb_20, emb_21, emb_22, emb_23, emb_24, emb_25, W, b):
    embs = [emb_0, emb_1, emb_2, emb_3, emb_4, emb_5, emb_6, emb_7, emb_8,
            emb_9, emb_10, emb_11, emb_12, emb_13, emb_14, emb_15, emb_16,
            emb_17, emb_18, emb_19, emb_20, emb_21, emb_22, emb_23, emb_24,
            emb_25]
    flat = [e.T.reshape(-1) for e in embs]
    cats = jnp.stack(
        [cat_0, cat_1, cat_2, cat_3, cat_4, cat_5, cat_6, cat_7, cat_8,
         cat_9, cat_10, cat_11, cat_12, cat_13, cat_14, cat_15, cat_16,
         cat_17, cat_18, cat_19, cat_20, cat_21, cat_22, cat_23, cat_24,
         cat_25], axis=0).astype(jnp.int32)
    dims = jnp.arange(D, dtype=jnp.int32)
    slots = []
    zero_slot = jnp.zeros((B, D), jnp.int32)
    for g in range(NG):
        grp = _GROUPS[g]
        for s in range(4):
            if s < len(grp):
                i = grp[s]
                v = jnp.int32(embs[i].shape[0])
                slots.append(cats[i][:, None] + dims[None, :] * v)
            else:
                slots.append(zero_slot)
    idxe = jnp.stack(slots).reshape(NG, 4, NW, NCHUNK, CHUNK, D).transpose(
        2, 0, 3, 1, 4, 5).reshape(NW, NG, NCHUNK, 4, 2048)
    num_wide = jnp.pad(
        numerical, ((0, 0), (0, NUM_PAD - NUM))).reshape(B // 8, 128)
    zeros = jnp.zeros((2, CHUNK, 128), jnp.float32)
    X = _sc_fill(zeros, num_wide, idxe, *flat)
    W_pad = jnp.concatenate(
        [W[:NUM], jnp.zeros((32 - NUM, H), W.dtype), W[NUM:],
         jnp.zeros((XW - 32 - NUM_TABLES * D, H), W.dtype)], axis=0)
    return _tc_matmul(X, W_pad, b.reshape(1, H))


# speedup vs baseline: 3.0969x; 3.0969x over previous
import functools

import jax
import jax.numpy as jnp
from jax import lax
from jax.experimental import pallas as pl
from jax.experimental.pallas import tpu as pltpu
from jax.experimental.pallas import tpu_sc as plsc

B = 16384
D = 32
NUM_TABLES = 26
NUM = 13
NUM_PAD = 16
H = 256
XW = 896

_info = plsc.get_sparse_core_info()
NC = _info.num_cores
NS = _info.num_subcores
NW = NC * NS
BPW = B // NW

PAD0 = NUM_PAD + NUM_TABLES * D


def _sc_body(zeros_hbm, num_hbm, idx_hbm, *rest):
    tables = rest[:NUM_TABLES]
    x_out = rest[NUM_TABLES]
    idx_all, rows, sem_g, sem_w0, sem_w1 = rest[NUM_TABLES + 1:]

    wid = lax.axis_index("s") * NC + lax.axis_index("c")
    base = wid * BPW

    pltpu.sync_copy(idx_hbm.at[wid], idx_all)
    pltpu.sync_copy(num_hbm.at[pl.ds(base, BPW)],
                    x_out.at[pl.ds(base, BPW), pl.ds(0, NUM_PAD)])
    pltpu.sync_copy(zeros_hbm,
                    x_out.at[pl.ds(base, BPW), pl.ds(PAD0, XW - PAD0)])

    sem_w = (sem_w0, sem_w1)
    pending = [None, None]
    for i in range(NUM_TABLES):
        p = i % 2
        if pending[p] is not None:
            pending[p].wait()
        g = pltpu.async_copy(tables[i].at[idx_all.at[i]], rows.at[p], sem_g)
        g.wait()
        pending[p] = pltpu.async_copy(
            rows.at[p],
            x_out.at[pl.ds(base, BPW), pl.ds(NUM_PAD + D * i, D)],
            sem_w[p],
        )
    pending[0].wait()
    pending[1].wait()


_sc_fill = functools.partial(
    pl.kernel,
    mesh=plsc.VectorSubcoreMesh(core_axis_name="c", subcore_axis_name="s"),
    out_type=jax.ShapeDtypeStruct((B, XW), jnp.float32),
    compiler_params=pltpu.CompilerParams(use_tc_tiling_on_sc=False),
    scratch_types=[
        pltpu.VMEM((NUM_TABLES, BPW), jnp.int32),
        pltpu.VMEM((2, BPW, D), jnp.float32),
        pltpu.SemaphoreType.DMA,
        pltpu.SemaphoreType.DMA,
        pltpu.SemaphoreType.DMA,
    ],
)(_sc_body)


TB = 1024


def _mm_body(x_ref, w_ref, b_ref, o_ref):
    o_ref[...] = (
        jnp.dot(x_ref[...], w_ref[...], preferred_element_type=jnp.float32)
        + b_ref[...]
    )


_tc_matmul = pl.pallas_call(
    _mm_body,
    grid=(B // TB,),
    in_specs=[
        pl.BlockSpec((TB, XW), lambda i: (i, 0)),
        pl.BlockSpec((XW, H), lambda i: (0, 0)),
        pl.BlockSpec((1, H), lambda i: (0, 0)),
    ],
    out_specs=pl.BlockSpec((TB, H), lambda i: (i, 0)),
    out_shape=jax.ShapeDtypeStruct((B, H), jnp.float32),
)


def kernel(numerical, cat_0, cat_1, cat_2, cat_3, cat_4, cat_5, cat_6, cat_7, cat_8, cat_9, cat_10, cat_11, cat_12, cat_13, cat_14, cat_15, cat_16, cat_17, cat_18, cat_19, cat_20, cat_21, cat_22, cat_23, cat_24, cat_25, emb_0, emb_1, emb_2, emb_3, emb_4, emb_5, emb_6, emb_7, emb_8, emb_9, emb_10, emb_11, emb_12, emb_13, emb_14, emb_15, emb_16, emb_17, emb_18, emb_19, emb_20, emb_21, emb_22, emb_23, emb_24, emb_25, W, b):
    cats = jnp.stack(
        [cat_0, cat_1, cat_2, cat_3, cat_4, cat_5, cat_6, cat_7, cat_8,
         cat_9, cat_10, cat_11, cat_12, cat_13, cat_14, cat_15, cat_16,
         cat_17, cat_18, cat_19, cat_20, cat_21, cat_22, cat_23, cat_24,
         cat_25], axis=0).astype(jnp.int32)
    idx = cats.reshape(NUM_TABLES, NW, BPW).transpose(1, 0, 2)
    num_pad = jnp.pad(numerical, ((0, 0), (0, NUM_PAD - NUM)))
    zeros = jnp.zeros((BPW, XW - PAD0), jnp.float32)
    X = _sc_fill(
        zeros, num_pad, idx,
        emb_0, emb_1, emb_2, emb_3, emb_4, emb_5, emb_6, emb_7, emb_8,
        emb_9, emb_10, emb_11, emb_12, emb_13, emb_14, emb_15, emb_16,
        emb_17, emb_18, emb_19, emb_20, emb_21, emb_22, emb_23, emb_24,
        emb_25)
    W_pad = jnp.concatenate(
        [W[:NUM], jnp.zeros((NUM_PAD - NUM, H), W.dtype), W[NUM:],
         jnp.zeros((XW - PAD0, H), W.dtype)], axis=0)
    return _tc_matmul(X, W_pad, b.reshape(1, H))


# speedup vs baseline: 3.2835x; 1.0602x over previous
import functools

import jax
import jax.numpy as jnp
from jax import lax
from jax.experimental import pallas as pl
from jax.experimental.pallas import tpu as pltpu
from jax.experimental.pallas import tpu_sc as plsc

B = 16384
D = 32
NUM_TABLES = 26
NUM = 13
NUM_PAD = 16
H = 256
XW = NUM_PAD + NUM_TABLES * D

_info = plsc.get_sparse_core_info()
NC = _info.num_cores
NS = _info.num_subcores
NW = NC * NS
BPW = B // NW

PAD0 = NUM_PAD + NUM_TABLES * D


def _sc_body(num_hbm, idx_hbm, *rest):
    tables = rest[:NUM_TABLES]
    x_out = rest[NUM_TABLES]
    idx_all, rows, sem_g, sem_w0, sem_w1 = rest[NUM_TABLES + 1:]

    wid = lax.axis_index("s") * NC + lax.axis_index("c")
    base = wid * BPW

    pltpu.sync_copy(idx_hbm.at[wid], idx_all)
    pltpu.sync_copy(num_hbm.at[pl.ds(base, BPW)],
                    x_out.at[pl.ds(base, BPW), pl.ds(0, NUM_PAD)])

    sem_w = (sem_w0, sem_w1)
    pending = [None, None]
    for i in range(NUM_TABLES):
        p = i % 2
        if pending[p] is not None:
            pending[p].wait()
        g = pltpu.async_copy(tables[i].at[idx_all.at[i]], rows.at[p], sem_g)
        g.wait()
        pending[p] = pltpu.async_copy(
            rows.at[p],
            x_out.at[pl.ds(base, BPW), pl.ds(NUM_PAD + D * i, D)],
            sem_w[p],
        )
    pending[0].wait()
    pending[1].wait()


_sc_fill = functools.partial(
    pl.kernel,
    mesh=plsc.VectorSubcoreMesh(core_axis_name="c", subcore_axis_name="s"),
    out_type=jax.ShapeDtypeStruct((B, XW), jnp.float32),
    compiler_params=pltpu.CompilerParams(use_tc_tiling_on_sc=False),
    scratch_types=[
        pltpu.VMEM((NUM_TABLES, BPW), jnp.int32),
        pltpu.VMEM((2, BPW, D), jnp.float32),
        pltpu.SemaphoreType.DMA,
        pltpu.SemaphoreType.DMA,
        pltpu.SemaphoreType.DMA,
    ],
)(_sc_body)


TB = 1024


def _mm_body(x_ref, w_ref, b_ref, o_ref):
    o_ref[...] = (
        jnp.dot(x_ref[...], w_ref[...], preferred_element_type=jnp.float32)
        + b_ref[...]
    )


_tc_matmul = pl.pallas_call(
    _mm_body,
    grid=(B // TB,),
    in_specs=[
        pl.BlockSpec((TB, XW), lambda i: (i, 0)),
        pl.BlockSpec((XW, H), lambda i: (0, 0)),
        pl.BlockSpec((1, H), lambda i: (0, 0)),
    ],
    out_specs=pl.BlockSpec((TB, H), lambda i: (i, 0)),
    out_shape=jax.ShapeDtypeStruct((B, H), jnp.float32),
)


def kernel(numerical, cat_0, cat_1, cat_2, cat_3, cat_4, cat_5, cat_6, cat_7, cat_8, cat_9, cat_10, cat_11, cat_12, cat_13, cat_14, cat_15, cat_16, cat_17, cat_18, cat_19, cat_20, cat_21, cat_22, cat_23, cat_24, cat_25, emb_0, emb_1, emb_2, emb_3, emb_4, emb_5, emb_6, emb_7, emb_8, emb_9, emb_10, emb_11, emb_12, emb_13, emb_14, emb_15, emb_16, emb_17, emb_18, emb_19, emb_20, emb_21, emb_22, emb_23, emb_24, emb_25, W, b):
    cats = jnp.stack(
        [cat_0, cat_1, cat_2, cat_3, cat_4, cat_5, cat_6, cat_7, cat_8,
         cat_9, cat_10, cat_11, cat_12, cat_13, cat_14, cat_15, cat_16,
         cat_17, cat_18, cat_19, cat_20, cat_21, cat_22, cat_23, cat_24,
         cat_25], axis=0).astype(jnp.int32)
    idx = cats.reshape(NUM_TABLES, NW, BPW).transpose(1, 0, 2)
    num_pad = jnp.pad(numerical, ((0, 0), (0, NUM_PAD - NUM)))
    X = _sc_fill(
        num_pad, idx,
        emb_0, emb_1, emb_2, emb_3, emb_4, emb_5, emb_6, emb_7, emb_8,
        emb_9, emb_10, emb_11, emb_12, emb_13, emb_14, emb_15, emb_16,
        emb_17, emb_18, emb_19, emb_20, emb_21, emb_22, emb_23, emb_24,
        emb_25)
    W_pad = jnp.concatenate(
        [W[:NUM], jnp.zeros((NUM_PAD - NUM, H), W.dtype), W[NUM:]], axis=0)
    return _tc_matmul(X, W_pad, b.reshape(1, H))


# speedup vs baseline: 3.3393x; 1.0170x over previous
import functools

import jax
import jax.numpy as jnp
from jax import lax
from jax.experimental import pallas as pl
from jax.experimental.pallas import tpu as pltpu
from jax.experimental.pallas import tpu_sc as plsc

B = 16384
D = 32
NUM_TABLES = 26
NUM = 13
NUM_PAD = 16
H = 256
XW = NUM_PAD + NUM_TABLES * D
NTA = 13
WA = NUM_PAD + NTA * D
WB = (NUM_TABLES - NTA) * D

_info = plsc.get_sparse_core_info()
NC = _info.num_cores
NS = _info.num_subcores
NW = NC * NS
BPW = B // NW


def _make_body(n_tables, with_num):
    def body(num_hbm, idx_hbm, *rest):
        tables = rest[:n_tables]
        x_out = rest[n_tables]
        idx_all, rows, sem_g, sem_w0, sem_w1 = rest[n_tables + 1:]

        wid = lax.axis_index("s") * NC + lax.axis_index("c")
        base = wid * BPW
        col0 = NUM_PAD if with_num else 0

        pltpu.sync_copy(idx_hbm.at[wid], idx_all)
        if with_num:
            pltpu.sync_copy(num_hbm.at[pl.ds(base, BPW)],
                            x_out.at[pl.ds(base, BPW), pl.ds(0, NUM_PAD)])

        sem_w = (sem_w0, sem_w1)
        pending = [None, None]
        for i in range(n_tables):
            p = i % 2
            if pending[p] is not None:
                pending[p].wait()
            g = pltpu.async_copy(tables[i].at[idx_all.at[i]],
                                 rows.at[p], sem_g)
            g.wait()
            pending[p] = pltpu.async_copy(
                rows.at[p],
                x_out.at[pl.ds(base, BPW), pl.ds(col0 + D * i, D)],
                sem_w[p],
            )
        pending[0].wait()
        pending[1].wait()

    return body


def _make_sc(n_tables, with_num, width):
    return functools.partial(
        pl.kernel,
        mesh=plsc.VectorSubcoreMesh(core_axis_name="c", subcore_axis_name="s"),
        out_type=jax.ShapeDtypeStruct((B, width), jnp.float32),
        compiler_params=pltpu.CompilerParams(use_tc_tiling_on_sc=False),
        scratch_types=[
            pltpu.VMEM((n_tables, BPW), jnp.int32),
            pltpu.VMEM((2, BPW, D), jnp.float32),
            pltpu.SemaphoreType.DMA,
            pltpu.SemaphoreType.DMA,
            pltpu.SemaphoreType.DMA,
        ],
    )(_make_body(n_tables, with_num))


_sc_a = _make_sc(NTA, True, WA)
_sc_b = _make_sc(NUM_TABLES - NTA, False, WB)


TB = 1024


def _mm_body(xa_ref, xb_ref, w_ref, b_ref, o_ref):
    x = jnp.concatenate([xa_ref[...], xb_ref[...]], axis=1)
    o_ref[...] = (
        jnp.dot(x, w_ref[...], preferred_element_type=jnp.float32)
        + b_ref[...]
    )


_tc_matmul = pl.pallas_call(
    _mm_body,
    grid=(B // TB,),
    in_specs=[
        pl.BlockSpec((TB, WA), lambda i: (i, 0)),
        pl.BlockSpec((TB, WB), lambda i: (i, 0)),
        pl.BlockSpec((XW, H), lambda i: (0, 0)),
        pl.BlockSpec((1, H), lambda i: (0, 0)),
    ],
    out_specs=pl.BlockSpec((TB, H), lambda i: (i, 0)),
    out_shape=jax.ShapeDtypeStruct((B, H), jnp.float32),
)


def kernel(numerical, cat_0, cat_1, cat_2, cat_3, cat_4, cat_5, cat_6, cat_7, cat_8, cat_9, cat_10, cat_11, cat_12, cat_13, cat_14, cat_15, cat_16, cat_17, cat_18, cat_19, cat_20, cat_21, cat_22, cat_23, cat_24, cat_25, emb_0, emb_1, emb_2, emb_3, emb_4, emb_5, emb_6, emb_7, emb_8, emb_9, emb_10, emb_11, emb_12, emb_13, emb_14, emb_15, emb_16, emb_17, emb_18, emb_19, emb_20, emb_21, emb_22, emb_23, emb_24, emb_25, W, b):
    embs = [emb_0, emb_1, emb_2, emb_3, emb_4, emb_5, emb_6, emb_7, emb_8,
            emb_9, emb_10, emb_11, emb_12, emb_13, emb_14, emb_15, emb_16,
            emb_17, emb_18, emb_19, emb_20, emb_21, emb_22, emb_23, emb_24,
            emb_25]
    cats = jnp.stack(
        [cat_0, cat_1, cat_2, cat_3, cat_4, cat_5, cat_6, cat_7, cat_8,
         cat_9, cat_10, cat_11, cat_12, cat_13, cat_14, cat_15, cat_16,
         cat_17, cat_18, cat_19, cat_20, cat_21, cat_22, cat_23, cat_24,
         cat_25], axis=0).astype(jnp.int32)
    idx = cats.reshape(NUM_TABLES, NW, BPW).transpose(1, 0, 2)
    num_pad = jnp.pad(numerical, ((0, 0), (0, NUM_PAD - NUM)))
    xa = _sc_a(num_pad, idx[:, :NTA], *embs[:NTA])
    xb = _sc_b(num_pad, idx[:, NTA:], *embs[NTA:])
    W_pad = jnp.concatenate(
        [W[:NUM], jnp.zeros((NUM_PAD - NUM, H), W.dtype), W[NUM:]], axis=0)
    return _tc_matmul(xa, xb, W_pad, b.reshape(1, H))


# speedup vs baseline: 3.4060x; 1.0200x over previous
import functools

import jax
import jax.numpy as jnp
from jax import lax
from jax.experimental import pallas as pl
from jax.experimental.pallas import tpu as pltpu
from jax.experimental.pallas import tpu_sc as plsc

B = 16384
D = 32
NUM_TABLES = 26
NUM = 13
NUM_PAD = 16
H = 256
XW = NUM_PAD + NUM_TABLES * D
ORDER_A = list(range(2, NUM_TABLES))
ORDER_B = [0, 1]
NTA = len(ORDER_A)
WA = NUM_PAD + NTA * D
WB = len(ORDER_B) * D

_info = plsc.get_sparse_core_info()
NC = _info.num_cores
NS = _info.num_subcores
NW = NC * NS
BPW = B // NW


def _make_body(n_tables, with_num):
    def body(num_hbm, idx_hbm, *rest):
        tables = rest[:n_tables]
        x_out = rest[n_tables]
        idx_all, rows, sem_g, sem_w0, sem_w1 = rest[n_tables + 1:]

        wid = lax.axis_index("s") * NC + lax.axis_index("c")
        base = wid * BPW
        col0 = NUM_PAD if with_num else 0

        pltpu.sync_copy(idx_hbm.at[wid], idx_all)
        if with_num:
            pltpu.sync_copy(num_hbm.at[pl.ds(base, BPW)],
                            x_out.at[pl.ds(base, BPW), pl.ds(0, NUM_PAD)])

        sem_w = (sem_w0, sem_w1)
        pending = [None, None]
        for i in range(n_tables):
            p = i % 2
            if pending[p] is not None:
                pending[p].wait()
            g = pltpu.async_copy(tables[i].at[idx_all.at[i]],
                                 rows.at[p], sem_g)
            g.wait()
            pending[p] = pltpu.async_copy(
                rows.at[p],
                x_out.at[pl.ds(base, BPW), pl.ds(col0 + D * i, D)],
                sem_w[p],
            )
        pending[0].wait()
        pending[1].wait()

    return body


def _make_sc(n_tables, with_num, width):
    return functools.partial(
        pl.kernel,
        mesh=plsc.VectorSubcoreMesh(core_axis_name="c", subcore_axis_name="s"),
        out_type=jax.ShapeDtypeStruct((B, width), jnp.float32),
        compiler_params=pltpu.CompilerParams(use_tc_tiling_on_sc=False),
        scratch_types=[
            pltpu.VMEM((n_tables, BPW), jnp.int32),
            pltpu.VMEM((2, BPW, D), jnp.float32),
            pltpu.SemaphoreType.DMA,
            pltpu.SemaphoreType.DMA,
            pltpu.SemaphoreType.DMA,
        ],
    )(_make_body(n_tables, with_num))


_sc_a = _make_sc(NTA, True, WA)
_sc_b = _make_sc(len(ORDER_B), False, WB)


TB = 1024


def _mm_body(xa_ref, xb_ref, w_ref, b_ref, o_ref):
    x = jnp.concatenate([xa_ref[...], xb_ref[...]], axis=1)
    o_ref[...] = (
        jnp.dot(x, w_ref[...], preferred_element_type=jnp.float32)
        + b_ref[...]
    )


_tc_matmul = pl.pallas_call(
    _mm_body,
    grid=(B // TB,),
    in_specs=[
        pl.BlockSpec((TB, WA), lambda i: (i, 0)),
        pl.BlockSpec((TB, WB), lambda i: (i, 0)),
        pl.BlockSpec((XW, H), lambda i: (0, 0)),
        pl.BlockSpec((1, H), lambda i: (0, 0)),
    ],
    out_specs=pl.BlockSpec((TB, H), lambda i: (i, 0)),
    out_shape=jax.ShapeDtypeStruct((B, H), jnp.float32),
)


def kernel(numerical, cat_0, cat_1, cat_2, cat_3, cat_4, cat_5, cat_6, cat_7, cat_8, cat_9, cat_10, cat_11, cat_12, cat_13, cat_14, cat_15, cat_16, cat_17, cat_18, cat_19, cat_20, cat_21, cat_22, cat_23, cat_24, cat_25, emb_0, emb_1, emb_2, emb_3, emb_4, emb_5, emb_6, emb_7, emb_8, emb_9, emb_10, emb_11, emb_12, emb_13, emb_14, emb_15, emb_16, emb_17, emb_18, emb_19, emb_20, emb_21, emb_22, emb_23, emb_24, emb_25, W, b):
    embs = [emb_0, emb_1, emb_2, emb_3, emb_4, emb_5, emb_6, emb_7, emb_8,
            emb_9, emb_10, emb_11, emb_12, emb_13, emb_14, emb_15, emb_16,
            emb_17, emb_18, emb_19, emb_20, emb_21, emb_22, emb_23, emb_24,
            emb_25]
    cats = jnp.stack(
        [cat_0, cat_1, cat_2, cat_3, cat_4, cat_5, cat_6, cat_7, cat_8,
         cat_9, cat_10, cat_11, cat_12, cat_13, cat_14, cat_15, cat_16,
         cat_17, cat_18, cat_19, cat_20, cat_21, cat_22, cat_23, cat_24,
         cat_25], axis=0).astype(jnp.int32)
    idx = cats.reshape(NUM_TABLES, NW, BPW).transpose(1, 0, 2)
    num_pad = jnp.pad(numerical, ((0, 0), (0, NUM_PAD - NUM)))
    xa = _sc_a(num_pad, idx[:, ORDER_A], *[embs[i] for i in ORDER_A])
    xb = _sc_b(num_pad, idx[:, ORDER_B], *[embs[i] for i in ORDER_B])
    W_pad = jnp.concatenate(
        [W[:NUM], jnp.zeros((NUM_PAD - NUM, H), W.dtype)]
        + [W[NUM + D * i: NUM + D * (i + 1)] for i in ORDER_A + ORDER_B],
        axis=0)
    return _tc_matmul(xa, xb, W_pad, b.reshape(1, H))
